# baseline (speedup 1.0000x reference)
import jax
import jax.numpy as jnp
from jax.experimental import pallas as pl
from jax.experimental.pallas import tpu as pltpu

_TOP_K = 8
_RC = 64


def _topk_softmax_chunk(s, iota, ef):
    vals = []
    idxs = []
    for k in range(_TOP_K):
        m = jnp.max(s, axis=1, keepdims=True)
        eq = s == m
        hit = jnp.where(eq, iota, ef)
        idx = jnp.min(hit, axis=1, keepdims=True)
        vals.append(m)
        idxs.append(idx)
        if k + 1 < _TOP_K:
            s = jnp.where(eq, -jnp.inf, s)
    v = jnp.concatenate(vals, axis=1)
    ix = jnp.concatenate(idxs, axis=1)
    ex = jnp.exp(v - v[:, 0:1])
    return ex / jnp.sum(ex, axis=1, keepdims=True), ix.astype(jnp.int32)


def _router_body(x_ref, w_ref, probs_ref, idx_ref, sa_ref, sb_ref):
    bt, e = sa_ref.shape
    w = w_ref[...]
    iota = jax.lax.broadcasted_iota(jnp.int32, (_RC, e), 1).astype(jnp.float32)
    ef = float(e)
    for c in range(bt // _RC):
        lo, hi = c * _RC, (c + 1) * _RC
        sa_ref[lo:hi, :] = jnp.dot(
            x_ref[lo:hi, :], w, preferred_element_type=jnp.float32)
        p, ix = _topk_softmax_chunk(sb_ref[lo:hi, :], iota, ef)
        probs_ref[lo:hi, :] = p
        idx_ref[lo:hi, :] = ix
    sb_ref[...] = sa_ref[...]


def kernel(x, W_gate):
    b, s, d = x.shape
    e = W_gate.shape[0]
    t = b * s
    xf = x.reshape(t, d)
    wt = W_gate.T
    bt = min(512, t)
    n = t // bt
    probs, idx = pl.pallas_call(
        _router_body,
        grid=(n + 1,),
        in_specs=[
            pl.BlockSpec((bt, d), lambda i: (jnp.minimum(i, n - 1), 0)),
            pl.BlockSpec((d, e), lambda i: (0, 0)),
        ],
        out_specs=[
            pl.BlockSpec((bt, _TOP_K), lambda i: (jnp.maximum(i - 1, 0), 0)),
            pl.BlockSpec((bt, _TOP_K), lambda i: (jnp.maximum(i - 1, 0), 0)),
        ],
        out_shape=[
            jax.ShapeDtypeStruct((t, _TOP_K), jnp.float32),
            jax.ShapeDtypeStruct((t, _TOP_K), jnp.int32),
        ],
        scratch_shapes=[
            pltpu.VMEM((bt, e), jnp.float32),
            pltpu.VMEM((bt, e), jnp.float32),
        ],
    )(xf, wt)
    return probs.reshape(b, s, _TOP_K), idx.reshape(b, s, _TOP_K)


# speedup vs baseline: 1.2014x; 1.2014x over previous
import jax
import jax.numpy as jnp
from jax.experimental import pallas as pl
from jax.experimental.pallas import tpu as pltpu

_TOP_K = 8


def _topk_softmax_chunk(s, iota, ef):
    vals = []
    idxs = []
    for k in range(_TOP_K):
        m = jnp.max(s, axis=1, keepdims=True)
        eq = s == m
        hit = jnp.where(eq, iota, ef)
        idx = jnp.min(hit, axis=1, keepdims=True)
        vals.append(m)
        idxs.append(idx)
        if k + 1 < _TOP_K:
            s = jnp.where(eq, -jnp.inf, s)
    v = jnp.concatenate(vals, axis=1)
    ix = jnp.concatenate(idxs, axis=1)
    ex = jnp.exp(v - v[:, 0:1])
    return ex / jnp.sum(ex, axis=1, keepdims=True), ix.astype(jnp.int32)


def _router_body(x_ref, w_ref, probs_ref, idx_ref):
    bt = x_ref.shape[0]
    e = w_ref.shape[1]
    h = bt // 2
    w = w_ref[...]
    rc = 64
    iota = jax.lax.broadcasted_iota(jnp.int32, (rc, e), 1).astype(jnp.float32)
    ef = float(e)
    s1 = jnp.dot(x_ref[0:h, :], w, preferred_element_type=jnp.float32)
    nchunks = h // rc
    s2_pieces = []
    out1 = []
    for c in range(nchunks):
        lo = h + c * rc
        s2_pieces.append(
            jnp.dot(x_ref[lo:lo + rc, :], w, preferred_element_type=jnp.float32))
        out1.append(_topk_softmax_chunk(s1[c * rc:(c + 1) * rc, :], iota, ef))
    for c in range(nchunks):
        p, ix = out1[c]
        probs_ref[c * rc:(c + 1) * rc, :] = p
        idx_ref[c * rc:(c + 1) * rc, :] = ix
        p2, ix2 = _topk_softmax_chunk(s2_pieces[c], iota, ef)
        lo = h + c * rc
        probs_ref[lo:lo + rc, :] = p2
        idx_ref[lo:lo + rc, :] = ix2


def kernel(x, W_gate):
    b, s, d = x.shape
    e = W_gate.shape[0]
    t = b * s
    xf = x.reshape(t, d)
    wt = W_gate.T
    bt = min(1024, t)
    grid = (t // bt,)
    probs, idx = pl.pallas_call(
        _router_body,
        grid=grid,
        in_specs=[
            pl.BlockSpec((bt, d), lambda i: (i, 0)),
            pl.BlockSpec((d, e), lambda i: (0, 0)),
        ],
        out_specs=[
            pl.BlockSpec((bt, _TOP_K), lambda i: (i, 0)),
            pl.BlockSpec((bt, _TOP_K), lambda i: (i, 0)),
        ],
        out_shape=[
            jax.ShapeDtypeStruct((t, _TOP_K), jnp.float32),
            jax.ShapeDtypeStruct((t, _TOP_K), jnp.int32),
        ],
        compiler_params=pltpu.CompilerParams(
            dimension_semantics=("parallel",)),
    )(xf, wt)
    return probs.reshape(b, s, _TOP_K), idx.reshape(b, s, _TOP_K)


# speedup vs baseline: 1.2429x; 1.0346x over previous
import jax
import jax.numpy as jnp
from jax.experimental import pallas as pl
from jax.experimental.pallas import tpu as pltpu

_TOP_K = 8


def _topk_softmax_chunk(s, iota, ef):
    vals = []
    idxs = []
    for k in range(_TOP_K):
        m = jnp.max(s, axis=1, keepdims=True)
        eq = s == m
        hit = jnp.where(eq, iota, ef)
        idx = jnp.min(hit, axis=1, keepdims=True)
        vals.append(m)
        idxs.append(idx)
        if k + 1 < _TOP_K:
            s = jnp.where(eq, -jnp.inf, s)
    v = jnp.concatenate(vals, axis=1)
    ix = jnp.concatenate(idxs, axis=1)
    ex = jnp.exp(v - v[:, 0:1])
    return ex / jnp.sum(ex, axis=1, keepdims=True), ix.astype(jnp.int32)


def _dot_wt(x, w):
    return jax.lax.dot_general(
        x, w, (((1,), (1,)), ((), ())), preferred_element_type=jnp.float32)


def _router_body(x_ref, w_ref, probs_ref, idx_ref):
    bt = x_ref.shape[0]
    e = w_ref.shape[0]
    h = bt // 2
    w = w_ref[...]
    rc = 64
    iota = jax.lax.broadcasted_iota(jnp.int32, (rc, e), 1).astype(jnp.float32)
    ef = float(e)
    s1 = _dot_wt(x_ref[0:h, :], w)
    nchunks = h // rc
    s2_pieces = []
    out1 = []
    for c in range(nchunks):
        lo = h + c * rc
        s2_pieces.append(_dot_wt(x_ref[lo:lo + rc, :], w))
        out1.append(_topk_softmax_chunk(s1[c * rc:(c + 1) * rc, :], iota, ef))
    for c in range(nchunks):
        p, ix = out1[c]
        probs_ref[c * rc:(c + 1) * rc, :] = p
        idx_ref[c * rc:(c + 1) * rc, :] = ix
        p2, ix2 = _topk_softmax_chunk(s2_pieces[c], iota, ef)
        lo = h + c * rc
        probs_ref[lo:lo + rc, :] = p2
        idx_ref[lo:lo + rc, :] = ix2


def kernel(x, W_gate):
    b, s, d = x.shape
    e = W_gate.shape[0]
    t = b * s
    xf = x.reshape(t, d)
    bt = min(1024, t)
    grid = (t // bt,)
    probs, idx = pl.pallas_call(
        _router_body,
        grid=grid,
        in_specs=[
            pl.BlockSpec((bt, d), lambda i: (i, 0)),
            pl.BlockSpec((e, d), lambda i: (0, 0)),
        ],
        out_specs=[
            pl.BlockSpec((bt, _TOP_K), lambda i: (i, 0)),
            pl.BlockSpec((bt, _TOP_K), lambda i: (i, 0)),
        ],
        out_shape=[
            jax.ShapeDtypeStruct((t, _TOP_K), jnp.float32),
            jax.ShapeDtypeStruct((t, _TOP_K), jnp.int32),
        ],
        compiler_params=pltpu.CompilerParams(
            dimension_semantics=("parallel",)),
    )(xf, W_gate)
    return probs.reshape(b, s, _TOP_K), idx.reshape(b, s, _TOP_K)
